# Initial kernel scaffold; baseline (speedup 1.0000x reference)
import jax
import jax.numpy as jnp
from jax.experimental import pallas as pl


def kernel(in_feat, edge_index, W, b):
    raise NotImplementedError("write your pallas kernel here")


# speedup vs baseline: 13.4516x; 13.4516x over previous
import functools

import jax
import jax.numpy as jnp
from jax import lax
from jax.experimental import pallas as pl
from jax.experimental.pallas import tpu as pltpu, tpu_sc as plsc

N = 100000
C = 16
F = 128
E = 3200000
ALPHA = 0.5
N_HOPS = 3

NS = 16
LANES = 16
N_P = 102400
RPT = N_P // NS
RCH = 256
NCH = RPT // RCH
TAIL = N % RCH

KW = 4
GRP = KW * 128
GPT = 391
E_P = NS * GPT * GRP
IDX_ROWS = E_P // 128
IROWS_PT = GPT * KW

PAD_IDX = N


def _rsqrt16(x):
    xb = lax.bitcast_convert_type(x, jnp.int32)
    y = lax.bitcast_convert_type(jnp.int32(0x5F3759DF) - (xb >> 1), jnp.float32)
    for _ in range(3):
        y = y * (1.5 - 0.5 * x * y * y)
    return y


def _sc_body(src_ref, dst_ref, h0_ref,
             out_ref, g0_ref, g_ref, nio_ref, nin_ref,
             acc, sidx, didx, rows, ones, accb, auxb, nscb, zbuf,
             gsem, ssem):
    tid = lax.axis_index("s")
    zeros16 = jnp.zeros((LANES,), jnp.float32)
    ones16 = jnp.ones((LANES,), jnp.float32)

    def _init(i, _):
        zbuf[i, :] = zeros16
        return 0
    lax.fori_loop(0, RCH, _init, 0)

    def _init1(i, _):
        ones[i, :] = ones16
        return 0
    lax.fori_loop(0, 128, _init1, 0)

    def _zero_all():
        def bd(c, _):
            pltpu.sync_copy(zbuf, acc.at[pl.ds(tid * RPT + c * RCH, RCH)])
            return 0
        lax.fori_loop(0, NCH, bd, 0)

    def _deg_pass(idx_hbm):
        def grp(g, _):
            base = tid * IROWS_PT + g * KW
            pltpu.sync_copy(idx_hbm.at[pl.ds(base, KW)], didx)
            descs = [pltpu.async_copy(ones, acc.at[didx.at[j]], ssem, add=True)
                     for j in range(KW)]
            for d in descs:
                d.wait()
            return 0
        lax.fori_loop(0, GPT, grp, 0)

    def _edge_pass(gsrc_ref):
        def grp(g, _):
            base = tid * IROWS_PT + g * KW
            pltpu.sync_copy(src_ref.at[pl.ds(base, KW)], sidx)
            pltpu.sync_copy(dst_ref.at[pl.ds(base, KW)], didx)
            gd = [pltpu.async_copy(gsrc_ref.at[sidx.at[j]], rows.at[j], gsem)
                  for j in range(KW)]
            for d in gd:
                d.wait()
            sd = [pltpu.async_copy(rows.at[j], acc.at[didx.at[j]], ssem, add=True)
                  for j in range(KW)]
            for d in sd:
                d.wait()
            return 0
        lax.fori_loop(0, GPT, grp, 0)

    def _phase_nout():
        def ch(c, _):
            base = tid * RPT + c * RCH
            pltpu.sync_copy(acc.at[pl.ds(base, RCH)], accb)

            def row(i, _):
                accb[i, :] = _rsqrt16(jnp.maximum(accb[i, :], 1.0))
                return 0
            lax.fori_loop(0, RCH, row, 0)
            pltpu.sync_copy(accb, nio_ref.at[pl.ds(base, RCH)])
            pltpu.sync_copy(zbuf, acc.at[pl.ds(base, RCH)])
            return 0
        lax.fori_loop(0, NCH, ch, 0)

    def _phase_nin_g0():
        def ch(c, _):
            base = tid * RPT + c * RCH
            pltpu.sync_copy(acc.at[pl.ds(base, RCH)], accb)
            pltpu.sync_copy(nio_ref.at[pl.ds(base, RCH)], nscb)
            pltpu.sync_copy(h0_ref.at[pl.ds(base, RCH)], auxb)

            def row(i, _):
                nin = _rsqrt16(jnp.maximum(accb[i, :], 1.0))
                auxb[i, :] = auxb[i, :] * nscb[i, :]
                nscb[i, :] = nin * nscb[i, :]
                accb[i, :] = nin
                return 0
            lax.fori_loop(0, RCH, row, 0)
            pltpu.sync_copy(auxb, g0_ref.at[pl.ds(base, RCH)])
            pltpu.sync_copy(accb, nin_ref.at[pl.ds(base, RCH)])
            pltpu.sync_copy(nscb, nio_ref.at[pl.ds(base, RCH)])
            pltpu.sync_copy(zbuf, acc.at[pl.ds(base, RCH)])
            return 0
        lax.fori_loop(0, NCH, ch, 0)

    def _combine(norm_hbm, aux_hbm, final):
        def ch(c, _):
            base = tid * RPT + c * RCH
            pltpu.sync_copy(acc.at[pl.ds(base, RCH)], accb)
            pltpu.sync_copy(aux_hbm.at[pl.ds(base, RCH)], auxb)
            pltpu.sync_copy(norm_hbm.at[pl.ds(base, RCH)], nscb)

            def row(i, _):
                accb[i, :] = ((1.0 - ALPHA) * nscb[i, :]) * accb[i, :] \
                    + ALPHA * auxb[i, :]
                return 0
            lax.fori_loop(0, RCH, row, 0)
            if final:
                full = base + RCH <= N
                part = jnp.logical_and(base < N, jnp.logical_not(full))

                @pl.when(full)
                def _():
                    pltpu.sync_copy(accb, out_ref.at[pl.ds(base, RCH)])

                @pl.when(part)
                def _():
                    pltpu.sync_copy(accb.at[pl.ds(0, TAIL)],
                                    out_ref.at[pl.ds(base, TAIL)])
            else:
                pltpu.sync_copy(accb, g_ref.at[pl.ds(base, RCH)])
                pltpu.sync_copy(zbuf, acc.at[pl.ds(base, RCH)])
            return 0
        lax.fori_loop(0, NCH, ch, 0)

    _zero_all()
    plsc.subcore_barrier()
    _deg_pass(src_ref)
    plsc.subcore_barrier()
    _phase_nout()
    plsc.subcore_barrier()
    _deg_pass(dst_ref)
    plsc.subcore_barrier()
    _phase_nin_g0()
    plsc.subcore_barrier()
    for hop in range(N_HOPS):
        _edge_pass(g0_ref if hop == 0 else g_ref)
        plsc.subcore_barrier()
        _combine(nio_ref if hop < N_HOPS - 1 else nin_ref,
                 g0_ref if hop < N_HOPS - 1 else h0_ref,
                 final=hop == N_HOPS - 1)
        plsc.subcore_barrier()


_sc_prop = functools.partial(
    pl.kernel,
    out_type=(
        jax.ShapeDtypeStruct((N, C), jnp.float32),
        jax.ShapeDtypeStruct((N_P, C), jnp.float32),
        jax.ShapeDtypeStruct((N_P, C), jnp.float32),
        jax.ShapeDtypeStruct((N_P, C), jnp.float32),
        jax.ShapeDtypeStruct((N_P, C), jnp.float32),
    ),
    mesh=plsc.VectorSubcoreMesh(core_axis_name="c", subcore_axis_name="s",
                                num_cores=1),
    compiler_params=pltpu.CompilerParams(use_tc_tiling_on_sc=False),
    scratch_types=[
        pltpu.VMEM_SHARED((N_P, C), jnp.float32),
        pltpu.VMEM((KW, 128), jnp.int32),
        pltpu.VMEM((KW, 128), jnp.int32),
        pltpu.VMEM((KW, 128, C), jnp.float32),
        pltpu.VMEM((128, C), jnp.float32),
        pltpu.VMEM((RCH, C), jnp.float32),
        pltpu.VMEM((RCH, C), jnp.float32),
        pltpu.VMEM((RCH, C), jnp.float32),
        pltpu.VMEM((RCH, C), jnp.float32),
        pltpu.SemaphoreType.DMA,
        pltpu.SemaphoreType.DMA,
    ],
)(_sc_body)


def _mm_body(x_ref, wt_ref, b_ref, o_ref):
    o_ref[...] = jnp.dot(x_ref[...], wt_ref[...],
                         preferred_element_type=jnp.float32) + b_ref[...]


def _mm(x, wt, b2):
    bm = 640
    return pl.pallas_call(
        _mm_body,
        grid=(N_P // bm,),
        in_specs=[pl.BlockSpec((bm, F), lambda m: (m, 0)),
                  pl.BlockSpec((F, C), lambda m: (0, 0)),
                  pl.BlockSpec((1, C), lambda m: (0, 0))],
        out_specs=pl.BlockSpec((bm, C), lambda m: (m, 0)),
        out_shape=jax.ShapeDtypeStruct((N_P, C), jnp.float32),
    )(x, wt, b2)


def kernel(in_feat, edge_index, W, b):
    src = edge_index[0].astype(jnp.int32)
    dst = edge_index[1].astype(jnp.int32)
    pad = jnp.full((E_P - E,), PAD_IDX, jnp.int32)
    src2 = jnp.concatenate([src, pad]).reshape(IDX_ROWS, 128)
    dst2 = jnp.concatenate([dst, pad]).reshape(IDX_ROWS, 128)
    xp = jnp.pad(in_feat, ((0, N_P - N), (0, 0)))
    h0 = _mm(xp, W.T, b[None, :])
    out = _sc_prop(src2, dst2, h0)[0]
    return out


# speedup vs baseline: 19.1870x; 1.4264x over previous
import functools

import jax
import jax.numpy as jnp
from jax import lax
from jax.experimental import pallas as pl
from jax.experimental.pallas import tpu as pltpu, tpu_sc as plsc

N = 100000
C = 16
F = 128
E = 3200000
ALPHA = 0.5
N_HOPS = 3

NS = 16
LANES = 16
N_P = 102400
RPT = N_P // NS
RCH = 128
NCH = RPT // RCH
TAIL = N % RCH

KW = 4
GRP = KW * 128
GPT = 392
E_P = NS * GPT * GRP
IDX_ROWS = E_P // 128
IROWS_PT = GPT * KW

PAD_IDX = N


def _rsqrt16(x):
    xb = lax.bitcast_convert_type(x, jnp.int32)
    y = lax.bitcast_convert_type(jnp.int32(0x5F3759DF) - (xb >> 1), jnp.float32)
    for _ in range(3):
        y = y * (1.5 - 0.5 * x * y * y)
    return y


def _sc_body(src_ref, dst_ref, h0_ref,
             out_ref, g0_ref, g_ref, nio_ref, nin_ref,
             acc, sidx, didx, rows, accb, auxb, nscb, zbuf,
             gsem, ssem, isem):
    tid = lax.axis_index("s")
    zeros16 = jnp.zeros((LANES,), jnp.float32)
    ones16 = jnp.ones((LANES,), jnp.float32)

    def _init(i, _):
        zbuf[i, :] = zeros16
        return 0
    lax.fori_loop(0, RCH, _init, 0)

    def _zero_all():
        def bd(c, _):
            pltpu.sync_copy(zbuf, acc.at[pl.ds(tid * RPT + c * RCH, RCH)])
            return 0
        lax.fori_loop(0, NCH, bd, 0)

    def _deg_pass(idx_hbm):
        ones_rows = rows.at[0].at[0]

        def fill(i, _):
            rows[0, 0, i, :] = ones16
            return 0
        lax.fori_loop(0, 128, fill, 0)

        def pair(p, _):
            base0 = tid * IROWS_PT + (2 * p) * KW
            i0 = pltpu.async_copy(idx_hbm.at[pl.ds(base0, KW)],
                                  didx.at[0], isem)
            i1 = pltpu.async_copy(idx_hbm.at[pl.ds(base0 + KW, KW)],
                                  didx.at[1], isem)
            i0.wait()
            s0 = [pltpu.async_copy(ones_rows, acc.at[didx.at[0].at[j]],
                                   ssem, add=True) for j in range(KW)]
            i1.wait()
            s1 = [pltpu.async_copy(ones_rows, acc.at[didx.at[1].at[j]],
                                   ssem, add=True) for j in range(KW)]
            for d in s0 + s1:
                d.wait()
            return 0
        lax.fori_loop(0, GPT // 2, pair, 0)

    def _edge_pass(gsrc_ref):
        def pair(p, _):
            base0 = tid * IROWS_PT + (2 * p) * KW
            base1 = base0 + KW
            is0 = pltpu.async_copy(src_ref.at[pl.ds(base0, KW)],
                                   sidx.at[0], isem)
            id0 = pltpu.async_copy(dst_ref.at[pl.ds(base0, KW)],
                                   didx.at[0], isem)
            is1 = pltpu.async_copy(src_ref.at[pl.ds(base1, KW)],
                                   sidx.at[1], isem)
            id1 = pltpu.async_copy(dst_ref.at[pl.ds(base1, KW)],
                                   didx.at[1], isem)
            is0.wait()
            g0 = [pltpu.async_copy(gsrc_ref.at[sidx.at[0].at[j]],
                                   rows.at[0].at[j], gsem) for j in range(KW)]
            is1.wait()
            g1 = [pltpu.async_copy(gsrc_ref.at[sidx.at[1].at[j]],
                                   rows.at[1].at[j], gsem) for j in range(KW)]
            id0.wait()
            for d in g0:
                d.wait()
            s0 = [pltpu.async_copy(rows.at[0].at[j], acc.at[didx.at[0].at[j]],
                                   ssem, add=True) for j in range(KW)]
            id1.wait()
            for d in g1:
                d.wait()
            s1 = [pltpu.async_copy(rows.at[1].at[j], acc.at[didx.at[1].at[j]],
                                   ssem, add=True) for j in range(KW)]
            for d in s0 + s1:
                d.wait()
            return 0
        lax.fori_loop(0, GPT // 2, pair, 0)

    def _phase_nout():
        def ch(c, _):
            base = tid * RPT + c * RCH
            pltpu.sync_copy(acc.at[pl.ds(base, RCH)], accb)

            def row(i, _):
                accb[i, :] = _rsqrt16(jnp.maximum(accb[i, :], 1.0))
                return 0
            lax.fori_loop(0, RCH, row, 0)
            pltpu.sync_copy(accb, nio_ref.at[pl.ds(base, RCH)])
            pltpu.sync_copy(zbuf, acc.at[pl.ds(base, RCH)])
            return 0
        lax.fori_loop(0, NCH, ch, 0)

    def _phase_nin_g0():
        def ch(c, _):
            base = tid * RPT + c * RCH
            pltpu.sync_copy(acc.at[pl.ds(base, RCH)], accb)
            pltpu.sync_copy(nio_ref.at[pl.ds(base, RCH)], nscb)
            pltpu.sync_copy(h0_ref.at[pl.ds(base, RCH)], auxb)

            def row(i, _):
                nin = _rsqrt16(jnp.maximum(accb[i, :], 1.0))
                auxb[i, :] = auxb[i, :] * nscb[i, :]
                nscb[i, :] = nin * nscb[i, :]
                accb[i, :] = nin
                return 0
            lax.fori_loop(0, RCH, row, 0)
            pltpu.sync_copy(auxb, g0_ref.at[pl.ds(base, RCH)])
            pltpu.sync_copy(accb, nin_ref.at[pl.ds(base, RCH)])
            pltpu.sync_copy(nscb, nio_ref.at[pl.ds(base, RCH)])
            pltpu.sync_copy(zbuf, acc.at[pl.ds(base, RCH)])
            return 0
        lax.fori_loop(0, NCH, ch, 0)

    def _combine(norm_hbm, aux_hbm, final):
        def ch(c, _):
            base = tid * RPT + c * RCH
            pltpu.sync_copy(acc.at[pl.ds(base, RCH)], accb)
            pltpu.sync_copy(aux_hbm.at[pl.ds(base, RCH)], auxb)
            pltpu.sync_copy(norm_hbm.at[pl.ds(base, RCH)], nscb)

            def row(i, _):
                accb[i, :] = ((1.0 - ALPHA) * nscb[i, :]) * accb[i, :] \
                    + ALPHA * auxb[i, :]
                return 0
            lax.fori_loop(0, RCH, row, 0)
            if final:
                full = base + RCH <= N
                part = jnp.logical_and(base < N, jnp.logical_not(full))

                @pl.when(full)
                def _():
                    pltpu.sync_copy(accb, out_ref.at[pl.ds(base, RCH)])

                @pl.when(part)
                def _():
                    pltpu.sync_copy(accb.at[pl.ds(0, TAIL)],
                                    out_ref.at[pl.ds(base, TAIL)])
            else:
                pltpu.sync_copy(accb, g_ref.at[pl.ds(base, RCH)])
                pltpu.sync_copy(zbuf, acc.at[pl.ds(base, RCH)])
            return 0
        lax.fori_loop(0, NCH, ch, 0)

    _zero_all()
    plsc.subcore_barrier()
    _deg_pass(src_ref)
    plsc.subcore_barrier()
    _phase_nout()
    plsc.subcore_barrier()
    _deg_pass(dst_ref)
    plsc.subcore_barrier()
    _phase_nin_g0()
    plsc.subcore_barrier()
    for hop in range(N_HOPS):
        _edge_pass(g0_ref if hop == 0 else g_ref)
        plsc.subcore_barrier()
        _combine(nio_ref if hop < N_HOPS - 1 else nin_ref,
                 g0_ref if hop < N_HOPS - 1 else h0_ref,
                 final=hop == N_HOPS - 1)
        plsc.subcore_barrier()


_sc_prop = functools.partial(
    pl.kernel,
    out_type=(
        jax.ShapeDtypeStruct((N, C), jnp.float32),
        jax.ShapeDtypeStruct((N_P, C), jnp.float32),
        jax.ShapeDtypeStruct((N_P, C), jnp.float32),
        jax.ShapeDtypeStruct((N_P, C), jnp.float32),
        jax.ShapeDtypeStruct((N_P, C), jnp.float32),
    ),
    mesh=plsc.VectorSubcoreMesh(core_axis_name="c", subcore_axis_name="s",
                                num_cores=1),
    compiler_params=pltpu.CompilerParams(use_tc_tiling_on_sc=False),
    scratch_types=[
        pltpu.VMEM_SHARED((N_P, C), jnp.float32),
        pltpu.VMEM((2, KW, 128), jnp.int32),
        pltpu.VMEM((2, KW, 128), jnp.int32),
        pltpu.VMEM((2, KW, 128, C), jnp.float32),
        pltpu.VMEM((RCH, C), jnp.float32),
        pltpu.VMEM((RCH, C), jnp.float32),
        pltpu.VMEM((RCH, C), jnp.float32),
        pltpu.VMEM((RCH, C), jnp.float32),
        pltpu.SemaphoreType.DMA,
        pltpu.SemaphoreType.DMA,
        pltpu.SemaphoreType.DMA,
    ],
)(_sc_body)


def _mm_body(x_ref, wt_ref, b_ref, o_ref):
    o_ref[...] = jnp.dot(x_ref[...], wt_ref[...],
                         preferred_element_type=jnp.float32) + b_ref[...]


def _mm(x, wt, b2):
    bm = 640
    return pl.pallas_call(
        _mm_body,
        grid=(N_P // bm,),
        in_specs=[pl.BlockSpec((bm, F), lambda m: (m, 0)),
                  pl.BlockSpec((F, C), lambda m: (0, 0)),
                  pl.BlockSpec((1, C), lambda m: (0, 0))],
        out_specs=pl.BlockSpec((bm, C), lambda m: (m, 0)),
        out_shape=jax.ShapeDtypeStruct((N_P, C), jnp.float32),
    )(x, wt, b2)


def kernel(in_feat, edge_index, W, b):
    src = edge_index[0].astype(jnp.int32)
    dst = edge_index[1].astype(jnp.int32)
    pad = jnp.full((E_P - E,), PAD_IDX, jnp.int32)
    src2 = jnp.concatenate([src, pad]).reshape(IDX_ROWS, 128)
    dst2 = jnp.concatenate([dst, pad]).reshape(IDX_ROWS, 128)
    xp = jnp.pad(in_feat, ((0, N_P - N), (0, 0)))
    h0 = _mm(xp, W.T, b[None, :])
    out = _sc_prop(src2, dst2, h0)[0]
    return out


# speedup vs baseline: 26.3629x; 1.3740x over previous
import functools

import jax
import jax.numpy as jnp
from jax import lax
from jax.experimental import pallas as pl
from jax.experimental.pallas import tpu as pltpu, tpu_sc as plsc

N = 100000
C = 16
F = 128
E = 3200000
ALPHA = 0.5
N_HOPS = 3

NS = 16
LANES = 16
N_P = 100352
RPT = N_P // NS
RCH = 128
NCH = RPT // RCH
TAIL = N % RCH

KW = 3
GRP = KW * 128
GPT = 522
E_P = NS * GPT * GRP
IDX_ROWS = E_P // 128
IROWS_PT = GPT * KW
LOOPN = (GPT + 3) // 3

PAD_IDX = N


def _rsqrt16(x):
    xb = lax.bitcast_convert_type(x, jnp.int32)
    y = lax.bitcast_convert_type(jnp.int32(0x5F3759DF) - (xb >> 1), jnp.float32)
    for _ in range(3):
        y = y * (1.5 - 0.5 * x * y * y)
    return y


def _sc_body(src_ref, dst_ref, h0_ref,
             out_ref, g0_ref, g_ref, nio_ref, nin_ref,
             acc, sidx, didx, rows, accb, auxb, nscb, zbuf,
             isem0, isem1, isem2, gsem0, gsem1, gsem2, ssem):
    tid = lax.axis_index("s")
    zeros16 = jnp.zeros((LANES,), jnp.float32)
    ones16 = jnp.ones((LANES,), jnp.float32)
    isem = [isem0, isem1, isem2]
    gsem = [gsem0, gsem1, gsem2]

    def _init(i, _):
        zbuf[i, :] = zeros16
        return 0
    lax.fori_loop(0, RCH, _init, 0)

    def _zero_all():
        def bd(c, _):
            pltpu.sync_copy(zbuf, acc.at[pl.ds(tid * RPT + c * RCH, RCH)])
            return 0
        lax.fori_loop(0, NCH, bd, 0)

    def _deg_pass(idx_hbm):
        def fill(i, _):
            for j in range(KW):
                rows[0, j, i, :] = ones16
            return 0
        lax.fori_loop(0, 128, fill, 0)
        ones_rows = rows.at[0]

        def outer(go, _):
            for b in range(3):
                g = go * 3 + b
                bp = (b - 1) % 3

                @pl.when(g >= 3)
                def _():
                    for j in range(KW):
                        pltpu.make_async_copy(
                            ones_rows.at[j], acc.at[didx.at[b].at[j]],
                            gsem[b]).wait()

                @pl.when(g < GPT)
                def _():
                    pltpu.async_copy(
                        idx_hbm.at[pl.ds(tid * IROWS_PT + g * KW, KW)],
                        didx.at[b], isem[b])

                @pl.when(jnp.logical_and(g >= 1, g <= GPT))
                def _():
                    pltpu.make_async_copy(
                        idx_hbm.at[pl.ds(tid * IROWS_PT + (g - 1) * KW, KW)],
                        didx.at[bp], isem[bp]).wait()
                    for j in range(KW):
                        pltpu.async_copy(ones_rows.at[j],
                                         acc.at[didx.at[bp].at[j]],
                                         gsem[bp], add=True)
            return 0
        lax.fori_loop(0, LOOPN, outer, 0)

    def _edge_pass(gsrc_ref):
        def outer(go, _):
            for b in range(3):
                g = go * 3 + b
                b1 = (b - 1) % 3
                b2 = (b - 2) % 3

                @pl.when(g >= 3)
                def _():
                    for j in range(KW):
                        pltpu.make_async_copy(
                            rows.at[b].at[j], acc.at[didx.at[b].at[j]],
                            ssem).wait()

                @pl.when(g < GPT)
                def _():
                    base = tid * IROWS_PT + g * KW
                    pltpu.async_copy(src_ref.at[pl.ds(base, KW)],
                                     sidx.at[b], isem[b])
                    pltpu.async_copy(dst_ref.at[pl.ds(base, KW)],
                                     didx.at[b], isem[b])

                @pl.when(jnp.logical_and(g >= 1, g <= GPT))
                def _():
                    base = tid * IROWS_PT + (g - 1) * KW
                    pltpu.make_async_copy(src_ref.at[pl.ds(base, KW)],
                                          sidx.at[b1], isem[b1]).wait()
                    pltpu.make_async_copy(dst_ref.at[pl.ds(base, KW)],
                                          didx.at[b1], isem[b1]).wait()
                    for j in range(KW):
                        pltpu.async_copy(gsrc_ref.at[sidx.at[b1].at[j]],
                                         rows.at[b1].at[j], gsem[b1])

                @pl.when(jnp.logical_and(g >= 2, g <= GPT + 1))
                def _():
                    for j in range(KW):
                        pltpu.make_async_copy(
                            gsrc_ref.at[sidx.at[b2].at[j]],
                            rows.at[b2].at[j], gsem[b2]).wait()
                    for j in range(KW):
                        pltpu.async_copy(rows.at[b2].at[j],
                                         acc.at[didx.at[b2].at[j]],
                                         ssem, add=True)
            return 0
        lax.fori_loop(0, LOOPN, outer, 0)

    def _phase_nout():
        def ch(c, _):
            base = tid * RPT + c * RCH
            pltpu.sync_copy(acc.at[pl.ds(base, RCH)], accb)

            def row(i, _):
                accb[i, :] = _rsqrt16(jnp.maximum(accb[i, :], 1.0))
                return 0
            lax.fori_loop(0, RCH, row, 0)
            pltpu.sync_copy(accb, nio_ref.at[pl.ds(base, RCH)])
            pltpu.sync_copy(zbuf, acc.at[pl.ds(base, RCH)])
            return 0
        lax.fori_loop(0, NCH, ch, 0)

    def _phase_nin_g0():
        def ch(c, _):
            base = tid * RPT + c * RCH
            a = pltpu.async_copy(acc.at[pl.ds(base, RCH)], accb, isem0)
            x = pltpu.async_copy(nio_ref.at[pl.ds(base, RCH)], nscb, isem1)
            h = pltpu.async_copy(h0_ref.at[pl.ds(base, RCH)], auxb, isem2)
            a.wait()
            x.wait()
            h.wait()

            def row(i, _):
                nin = _rsqrt16(jnp.maximum(accb[i, :], 1.0))
                auxb[i, :] = auxb[i, :] * nscb[i, :]
                nscb[i, :] = nin * nscb[i, :]
                accb[i, :] = nin
                return 0
            lax.fori_loop(0, RCH, row, 0)
            pltpu.sync_copy(auxb, g0_ref.at[pl.ds(base, RCH)])
            pltpu.sync_copy(accb, nin_ref.at[pl.ds(base, RCH)])
            pltpu.sync_copy(nscb, nio_ref.at[pl.ds(base, RCH)])
            pltpu.sync_copy(zbuf, acc.at[pl.ds(base, RCH)])
            return 0
        lax.fori_loop(0, NCH, ch, 0)

    def _combine(norm_hbm, aux_hbm, final):
        def ch(c, _):
            base = tid * RPT + c * RCH
            a = pltpu.async_copy(acc.at[pl.ds(base, RCH)], accb, isem0)
            x = pltpu.async_copy(aux_hbm.at[pl.ds(base, RCH)], auxb, isem1)
            n = pltpu.async_copy(norm_hbm.at[pl.ds(base, RCH)], nscb, isem2)
            a.wait()
            x.wait()
            n.wait()

            def row(i, _):
                accb[i, :] = ((1.0 - ALPHA) * nscb[i, :]) * accb[i, :] \
                    + ALPHA * auxb[i, :]
                return 0
            lax.fori_loop(0, RCH, row, 0)
            if final:
                full = base + RCH <= N
                part = jnp.logical_and(base < N, jnp.logical_not(full))

                @pl.when(full)
                def _():
                    pltpu.sync_copy(accb, out_ref.at[pl.ds(base, RCH)])

                @pl.when(part)
                def _():
                    pltpu.sync_copy(accb.at[pl.ds(0, TAIL)],
                                    out_ref.at[pl.ds(base, TAIL)])
            else:
                pltpu.sync_copy(accb, g_ref.at[pl.ds(base, RCH)])
                pltpu.sync_copy(zbuf, acc.at[pl.ds(base, RCH)])
            return 0
        lax.fori_loop(0, NCH, ch, 0)

    _zero_all()
    plsc.subcore_barrier()
    _deg_pass(src_ref)
    plsc.subcore_barrier()
    _phase_nout()
    plsc.subcore_barrier()
    _deg_pass(dst_ref)
    plsc.subcore_barrier()
    _phase_nin_g0()
    plsc.subcore_barrier()
    for hop in range(N_HOPS):
        _edge_pass(g0_ref if hop == 0 else g_ref)
        plsc.subcore_barrier()
        _combine(nio_ref if hop < N_HOPS - 1 else nin_ref,
                 g0_ref if hop < N_HOPS - 1 else h0_ref,
                 final=hop == N_HOPS - 1)
        plsc.subcore_barrier()


_sc_prop = functools.partial(
    pl.kernel,
    out_type=(
        jax.ShapeDtypeStruct((N, C), jnp.float32),
        jax.ShapeDtypeStruct((N_P, C), jnp.float32),
        jax.ShapeDtypeStruct((N_P, C), jnp.float32),
        jax.ShapeDtypeStruct((N_P, C), jnp.float32),
        jax.ShapeDtypeStruct((N_P, C), jnp.float32),
    ),
    mesh=plsc.VectorSubcoreMesh(core_axis_name="c", subcore_axis_name="s",
                                num_cores=1),
    compiler_params=pltpu.CompilerParams(use_tc_tiling_on_sc=False),
    scratch_types=[
        pltpu.VMEM_SHARED((N_P, C), jnp.float32),
        pltpu.VMEM((3, KW, 128), jnp.int32),
        pltpu.VMEM((3, KW, 128), jnp.int32),
        pltpu.VMEM((3, KW, 128, C), jnp.float32),
        pltpu.VMEM((RCH, C), jnp.float32),
        pltpu.VMEM((RCH, C), jnp.float32),
        pltpu.VMEM((RCH, C), jnp.float32),
        pltpu.VMEM((RCH, C), jnp.float32),
        pltpu.SemaphoreType.DMA,
        pltpu.SemaphoreType.DMA,
        pltpu.SemaphoreType.DMA,
        pltpu.SemaphoreType.DMA,
        pltpu.SemaphoreType.DMA,
        pltpu.SemaphoreType.DMA,
        pltpu.SemaphoreType.DMA,
    ],
)(_sc_body)


def _mm_body(x_ref, wt_ref, b_ref, o_ref):
    o_ref[...] = jnp.dot(x_ref[...], wt_ref[...],
                         preferred_element_type=jnp.float32) + b_ref[...]


def _mm(x, wt, b2):
    bm = 512
    return pl.pallas_call(
        _mm_body,
        grid=(N_P // bm,),
        in_specs=[pl.BlockSpec((bm, F), lambda m: (m, 0)),
                  pl.BlockSpec((F, C), lambda m: (0, 0)),
                  pl.BlockSpec((1, C), lambda m: (0, 0))],
        out_specs=pl.BlockSpec((bm, C), lambda m: (m, 0)),
        out_shape=jax.ShapeDtypeStruct((N_P, C), jnp.float32),
    )(x, wt, b2)


def kernel(in_feat, edge_index, W, b):
    src = edge_index[0].astype(jnp.int32)
    dst = edge_index[1].astype(jnp.int32)
    pad = jnp.full((E_P - E,), PAD_IDX, jnp.int32)
    src2 = jnp.concatenate([src, pad]).reshape(IDX_ROWS, 128)
    dst2 = jnp.concatenate([dst, pad]).reshape(IDX_ROWS, 128)
    xp = jnp.pad(in_feat, ((0, N_P - N), (0, 0)))
    h0 = _mm(xp, W.T, b[None, :])
    out = _sc_prop(src2, dst2, h0)[0]
    return out


# speedup vs baseline: 32.5678x; 1.2354x over previous
import functools

import jax
import jax.numpy as jnp
from jax import lax
from jax.experimental import pallas as pl
from jax.experimental.pallas import tpu as pltpu, tpu_sc as plsc

N = 100000
C = 16
F = 128
E = 3200000
ALPHA = 0.5

NS = 16
LANES = 16
N_P = 100352
RPT = N_P // NS
RCH = 128
NCH = RPT // RCH

RPT32 = N_P // 32
RCH2 = 112
NCH2 = RPT32 // RCH2
TAIL2 = 96

KW = 4
GRP = KW * 128
GPT1 = 396
GPT2 = 198
E_P = 16 * GPT1 * GRP
IDX_ROWS = E_P // 128

PAD_IDX = N

_MESH = plsc.VectorSubcoreMesh(core_axis_name="c", subcore_axis_name="s")
_PARAMS = pltpu.CompilerParams(use_tc_tiling_on_sc=False)
_F32 = jnp.float32


def _rsqrt16(x):
    xb = lax.bitcast_convert_type(x, jnp.int32)
    y = lax.bitcast_convert_type(jnp.int32(0x5F3759DF) - (xb >> 1), _F32)
    for _ in range(3):
        y = y * (1.5 - 0.5 * x * y * y)
    return y


def _fill_zbuf(zbuf, n):
    zeros16 = jnp.zeros((LANES,), _F32)

    def bd(i, _):
        zbuf[i, :] = zeros16
        return 0
    lax.fori_loop(0, n, bd, 0)


def _zero_acc(acc, zbuf, tid):
    def bd(c, _):
        pltpu.sync_copy(zbuf, acc.at[pl.ds(tid * RPT + c * RCH, RCH)])
        return 0
    lax.fori_loop(0, NCH, bd, 0)


def _deg_ring(idx_hbm, acc, didx, rows, isem, gsem, tid, gpt):
    ones16 = jnp.ones((LANES,), _F32)

    def fill(i, _):
        for j in range(KW):
            rows[0, j, i, :] = ones16
        return 0
    lax.fori_loop(0, 128, fill, 0)
    ones_rows = rows.at[0]
    irows_pt = gpt * KW

    def outer(go, _):
        for b in range(3):
            g = go * 3 + b
            bp = (b - 1) % 3

            @pl.when(g >= 3)
            def _():
                for j in range(KW):
                    pltpu.make_async_copy(
                        ones_rows.at[j], acc.at[didx.at[b].at[j]],
                        gsem[b]).wait()

            @pl.when(g < gpt)
            def _():
                pltpu.async_copy(
                    idx_hbm.at[pl.ds(tid * irows_pt + g * KW, KW)],
                    didx.at[b], isem[b])

            @pl.when(jnp.logical_and(g >= 1, g <= gpt))
            def _():
                pltpu.make_async_copy(
                    idx_hbm.at[pl.ds(tid * irows_pt + (g - 1) * KW, KW)],
                    didx.at[bp], isem[bp]).wait()
                for j in range(KW):
                    pltpu.async_copy(ones_rows.at[j],
                                     acc.at[didx.at[bp].at[j]],
                                     gsem[bp], add=True)
        return 0
    lax.fori_loop(0, (gpt + 3) // 3, outer, 0)


def _edge_ring(src_ref, dst_ref, gsrc_ref, acc, sidx, didx, rows,
               isem, gsem, ssem, ft, gpt):
    irows_pt = gpt * KW

    def outer(go, _):
        for b in range(3):
            g = go * 3 + b
            b1 = (b - 1) % 3
            b2 = (b - 2) % 3

            @pl.when(g >= 3)
            def _():
                for j in range(KW):
                    pltpu.make_async_copy(
                        rows.at[b].at[j], acc.at[didx.at[b].at[j]],
                        ssem).wait()

            @pl.when(g < gpt)
            def _():
                base = ft * irows_pt + g * KW
                pltpu.async_copy(src_ref.at[pl.ds(base, KW)],
                                 sidx.at[b], isem[b])
                pltpu.async_copy(dst_ref.at[pl.ds(base, KW)],
                                 didx.at[b], isem[b])

            @pl.when(jnp.logical_and(g >= 1, g <= gpt))
            def _():
                base = ft * irows_pt + (g - 1) * KW
                pltpu.make_async_copy(src_ref.at[pl.ds(base, KW)],
                                      sidx.at[b1], isem[b1]).wait()
                pltpu.make_async_copy(dst_ref.at[pl.ds(base, KW)],
                                      didx.at[b1], isem[b1]).wait()
                for j in range(KW):
                    pltpu.async_copy(gsrc_ref.at[sidx.at[b1].at[j]],
                                     rows.at[b1].at[j], gsem[b1])

            @pl.when(jnp.logical_and(g >= 2, g <= gpt + 1))
            def _():
                for j in range(KW):
                    pltpu.make_async_copy(
                        gsrc_ref.at[sidx.at[b2].at[j]],
                        rows.at[b2].at[j], gsem[b2]).wait()
                for j in range(KW):
                    pltpu.async_copy(rows.at[b2].at[j],
                                     acc.at[didx.at[b2].at[j]],
                                     ssem, add=True)
        return 0
    lax.fori_loop(0, (gpt + 3) // 3, outer, 0)


def _l1_body(src_ref, dst_ref, nout_ref, nin_ref,
             acc, didx, rows, accb, zbuf,
             isem0, isem1, isem2, gsem0, gsem1, gsem2):
    cid = lax.axis_index("c")
    tid = lax.axis_index("s")
    isem = [isem0, isem1, isem2]
    gsem = [gsem0, gsem1, gsem2]
    _fill_zbuf(zbuf, RCH)
    _zero_acc(acc, zbuf, tid)
    plsc.subcore_barrier()

    @pl.when(cid == 0)
    def _():
        _deg_ring(src_ref, acc, didx, rows, isem, gsem, tid, GPT1)

    @pl.when(cid == 1)
    def _():
        _deg_ring(dst_ref, acc, didx, rows, isem, gsem, tid, GPT1)
    plsc.subcore_barrier()

    def norm_phase(out_ref):
        def ch(c, _):
            base = tid * RPT + c * RCH
            pltpu.sync_copy(acc.at[pl.ds(base, RCH)], accb)

            def row(i, _):
                accb[i, :] = _rsqrt16(jnp.maximum(accb[i, :], 1.0))
                return 0
            lax.fori_loop(0, RCH, row, 0)
            pltpu.sync_copy(accb, out_ref.at[pl.ds(base, RCH)])
            return 0
        lax.fori_loop(0, NCH, ch, 0)

    @pl.when(cid == 0)
    def _():
        norm_phase(nout_ref)

    @pl.when(cid == 1)
    def _():
        norm_phase(nin_ref)


_l1 = functools.partial(
    pl.kernel,
    out_type=(jax.ShapeDtypeStruct((N_P, C), _F32),
              jax.ShapeDtypeStruct((N_P, C), _F32)),
    mesh=_MESH, compiler_params=_PARAMS,
    scratch_types=[
        pltpu.VMEM_SHARED((N_P, C), _F32),
        pltpu.VMEM((3, KW, 128), jnp.int32),
        pltpu.VMEM((3, KW, 128, C), _F32),
        pltpu.VMEM((RCH, C), _F32),
        pltpu.VMEM((RCH, C), _F32),
        pltpu.SemaphoreType.DMA, pltpu.SemaphoreType.DMA,
        pltpu.SemaphoreType.DMA, pltpu.SemaphoreType.DMA,
        pltpu.SemaphoreType.DMA, pltpu.SemaphoreType.DMA,
    ],
)(_l1_body)


def _l2_body(nout_ref, nin_ref, h0_ref, nio_ref, g0_ref,
             ab, bb, cb, sem0, sem1, sem2):
    ft = lax.axis_index("c") * NS + lax.axis_index("s")

    def ch(c, _):
        base = ft * RPT32 + c * RCH2
        d0 = pltpu.async_copy(nout_ref.at[pl.ds(base, RCH2)], ab, sem0)
        d1 = pltpu.async_copy(nin_ref.at[pl.ds(base, RCH2)], bb, sem1)
        d2 = pltpu.async_copy(h0_ref.at[pl.ds(base, RCH2)], cb, sem2)
        d0.wait()
        d1.wait()
        d2.wait()

        def row(i, _):
            cb[i, :] = cb[i, :] * ab[i, :]
            bb[i, :] = bb[i, :] * ab[i, :]
            return 0
        lax.fori_loop(0, RCH2, row, 0)
        full = base + RCH2 <= N
        part = jnp.logical_and(base < N, jnp.logical_not(full))

        @pl.when(full)
        def _():
            pltpu.sync_copy(cb, g0_ref.at[pl.ds(base, RCH2)])
            pltpu.sync_copy(bb, nio_ref.at[pl.ds(base, RCH2)])

        @pl.when(part)
        def _():
            pltpu.sync_copy(cb.at[pl.ds(0, TAIL2)],
                            g0_ref.at[pl.ds(base, TAIL2)])
            pltpu.sync_copy(bb.at[pl.ds(0, TAIL2)],
                            nio_ref.at[pl.ds(base, TAIL2)])
        return 0
    lax.fori_loop(0, NCH2, ch, 0)


_l2 = functools.partial(
    pl.kernel,
    out_type=(jax.ShapeDtypeStruct((N_P, C), _F32),
              jax.ShapeDtypeStruct((N_P, C), _F32)),
    mesh=_MESH, compiler_params=_PARAMS,
    scratch_types=[
        pltpu.VMEM((RCH2, C), _F32), pltpu.VMEM((RCH2, C), _F32),
        pltpu.VMEM((RCH2, C), _F32),
        pltpu.SemaphoreType.DMA, pltpu.SemaphoreType.DMA,
        pltpu.SemaphoreType.DMA,
    ],
)(_l2_body)


def _lh_body(src_ref, dst_ref, gsrc_ref, accp_ref,
             acc, sidx, didx, rows, zbuf,
             isem0, isem1, isem2, gsem0, gsem1, gsem2, ssem):
    cid = lax.axis_index("c")
    tid = lax.axis_index("s")
    ft = cid * NS + tid
    isem = [isem0, isem1, isem2]
    gsem = [gsem0, gsem1, gsem2]
    _fill_zbuf(zbuf, RCH)
    _zero_acc(acc, zbuf, tid)
    plsc.subcore_barrier()
    _edge_ring(src_ref, dst_ref, gsrc_ref, acc, sidx, didx, rows,
               isem, gsem, ssem, ft, GPT2)
    plsc.subcore_barrier()
    pltpu.sync_copy(acc.at[pl.ds(tid * RPT, RPT)],
                    accp_ref.at[cid].at[pl.ds(tid * RPT, RPT)])


_lh = functools.partial(
    pl.kernel,
    out_type=jax.ShapeDtypeStruct((2, N_P, C), _F32),
    mesh=_MESH, compiler_params=_PARAMS,
    scratch_types=[
        pltpu.VMEM_SHARED((N_P, C), _F32),
        pltpu.VMEM((3, KW, 128), jnp.int32),
        pltpu.VMEM((3, KW, 128), jnp.int32),
        pltpu.VMEM((3, KW, 128, C), _F32),
        pltpu.VMEM((RCH, C), _F32),
        pltpu.SemaphoreType.DMA, pltpu.SemaphoreType.DMA,
        pltpu.SemaphoreType.DMA, pltpu.SemaphoreType.DMA,
        pltpu.SemaphoreType.DMA, pltpu.SemaphoreType.DMA,
        pltpu.SemaphoreType.DMA,
    ],
)(_lh_body)


def _combine_body(final, accp_ref, norm_ref, aux_ref, out_ref,
                  ab, bb, cb, db, sem0, sem1, sem2, sem3):
    ft = lax.axis_index("c") * NS + lax.axis_index("s")

    def ch(c, _):
        base = ft * RPT32 + c * RCH2
        d0 = pltpu.async_copy(accp_ref.at[0].at[pl.ds(base, RCH2)], ab, sem0)
        d1 = pltpu.async_copy(accp_ref.at[1].at[pl.ds(base, RCH2)], bb, sem1)
        d2 = pltpu.async_copy(norm_ref.at[pl.ds(base, RCH2)], cb, sem2)
        d3 = pltpu.async_copy(aux_ref.at[pl.ds(base, RCH2)], db, sem3)
        d0.wait()
        d1.wait()
        d2.wait()
        d3.wait()

        def row(i, _):
            ab[i, :] = ((1.0 - ALPHA) * cb[i, :]) * (ab[i, :] + bb[i, :]) \
                + ALPHA * db[i, :]
            return 0
        lax.fori_loop(0, RCH2, row, 0)
        full = base + RCH2 <= N
        part = jnp.logical_and(base < N, jnp.logical_not(full))

        @pl.when(full)
        def _():
            pltpu.sync_copy(ab, out_ref.at[pl.ds(base, RCH2)])

        @pl.when(part)
        def _():
            pltpu.sync_copy(ab.at[pl.ds(0, TAIL2)],
                            out_ref.at[pl.ds(base, TAIL2)])
        return 0
    lax.fori_loop(0, NCH2, ch, 0)


def _mk_combine(final):
    shape = (N, C) if final else (N_P, C)
    return functools.partial(
        pl.kernel,
        out_type=jax.ShapeDtypeStruct(shape, _F32),
        mesh=_MESH, compiler_params=_PARAMS,
        scratch_types=[
            pltpu.VMEM((RCH2, C), _F32), pltpu.VMEM((RCH2, C), _F32),
            pltpu.VMEM((RCH2, C), _F32), pltpu.VMEM((RCH2, C), _F32),
            pltpu.SemaphoreType.DMA, pltpu.SemaphoreType.DMA,
            pltpu.SemaphoreType.DMA, pltpu.SemaphoreType.DMA,
        ],
    )(functools.partial(_combine_body, final))


_lc = _mk_combine(False)
_lf = _mk_combine(True)


def _mm_body(x_ref, wt_ref, b_ref, o_ref):
    o_ref[...] = jnp.dot(x_ref[...], wt_ref[...],
                         preferred_element_type=_F32) + b_ref[...]


def _mm(x, wt, b2):
    bm = 512
    return pl.pallas_call(
        _mm_body,
        grid=(N_P // bm,),
        in_specs=[pl.BlockSpec((bm, F), lambda m: (m, 0)),
                  pl.BlockSpec((F, C), lambda m: (0, 0)),
                  pl.BlockSpec((1, C), lambda m: (0, 0))],
        out_specs=pl.BlockSpec((bm, C), lambda m: (m, 0)),
        out_shape=jax.ShapeDtypeStruct((N_P, C), _F32),
    )(x, wt, b2)


def kernel(in_feat, edge_index, W, b):
    src = edge_index[0].astype(jnp.int32)
    dst = edge_index[1].astype(jnp.int32)
    pad = jnp.full((E_P - E,), PAD_IDX, jnp.int32)
    src2 = jnp.concatenate([src, pad]).reshape(IDX_ROWS, 128)
    dst2 = jnp.concatenate([dst, pad]).reshape(IDX_ROWS, 128)
    xp = jnp.pad(in_feat, ((0, N_P - N), (0, 0)))
    h0 = _mm(xp, W.T, b[None, :])
    nout, nin = _l1(src2, dst2)
    nio, g0 = _l2(nout, nin, h0)
    g = g0
    for hop in range(3):
        accp = _lh(src2, dst2, g)
        if hop < 2:
            g = _lc(accp, nio, g0)
        else:
            return _lf(accp, nin, h0)


# speedup vs baseline: 36.0400x; 1.1066x over previous
import functools

import jax
import jax.numpy as jnp
from jax import lax
from jax.experimental import pallas as pl
from jax.experimental.pallas import tpu as pltpu, tpu_sc as plsc

N = 100000
C = 16
F = 128
E = 3200000
ALPHA = 0.5

NS = 16
LANES = 16
N_P = 100352
RPT = N_P // NS
RCH = 128
NCH = RPT // RCH

RPT32 = N_P // 32
RCH2 = 112
NCH2 = RPT32 // RCH2
TAIL2 = 96

KW = 4
GRP = KW * 128
GPT1 = 396
GPT_A = 276
GPT_B = 120
E_P = 16 * GPT1 * GRP
IDX_ROWS = E_P // 128

PAD_IDX = N

_MESH = plsc.VectorSubcoreMesh(core_axis_name="c", subcore_axis_name="s")
_PARAMS = pltpu.CompilerParams(use_tc_tiling_on_sc=False)
_F32 = jnp.float32


def _rsqrt16(x):
    xb = lax.bitcast_convert_type(x, jnp.int32)
    y = lax.bitcast_convert_type(jnp.int32(0x5F3759DF) - (xb >> 1), _F32)
    for _ in range(3):
        y = y * (1.5 - 0.5 * x * y * y)
    return y


def _fill_zbuf(zbuf, n):
    zeros16 = jnp.zeros((LANES,), _F32)

    def bd(i, _):
        zbuf[i, :] = zeros16
        return 0
    lax.fori_loop(0, n, bd, 0)


def _zero_acc(acc, zbuf, tid):
    def bd(c, _):
        pltpu.sync_copy(zbuf, acc.at[pl.ds(tid * RPT + c * RCH, RCH)])
        return 0
    lax.fori_loop(0, NCH, bd, 0)


def _deg_ring(idx_hbm, acc, didx, rows, isem, gsem, tid, gpt):
    ones16 = jnp.ones((LANES,), _F32)

    def fill(i, _):
        for j in range(KW):
            rows[0, j, i, :] = ones16
        return 0
    lax.fori_loop(0, 128, fill, 0)
    ones_rows = rows.at[0]
    irows_pt = gpt * KW

    def outer(go, _):
        for b in range(3):
            g = go * 3 + b
            bp = (b - 1) % 3

            @pl.when(g >= 3)
            def _():
                for j in range(KW):
                    pltpu.make_async_copy(
                        ones_rows.at[j], acc.at[didx.at[b].at[j]],
                        gsem[b]).wait()

            @pl.when(g < gpt)
            def _():
                pltpu.async_copy(
                    idx_hbm.at[pl.ds(tid * irows_pt + g * KW, KW)],
                    didx.at[b], isem[b])

            @pl.when(jnp.logical_and(g >= 1, g <= gpt))
            def _():
                pltpu.make_async_copy(
                    idx_hbm.at[pl.ds(tid * irows_pt + (g - 1) * KW, KW)],
                    didx.at[bp], isem[bp]).wait()
                for j in range(KW):
                    pltpu.async_copy(ones_rows.at[j],
                                     acc.at[didx.at[bp].at[j]],
                                     gsem[bp], add=True)
        return 0
    lax.fori_loop(0, (gpt + 3) // 3, outer, 0)


def _edge_ring(src_ref, dst_ref, gsrc_ref, acc, sidx, didx, rows,
               isem, gsem, ssem, gbase, gpt):

    def outer(go, _):
        for b in range(3):
            g = go * 3 + b
            b1 = (b - 1) % 3
            b2 = (b - 2) % 3

            @pl.when(g >= 3)
            def _():
                for j in range(KW):
                    pltpu.make_async_copy(
                        rows.at[b].at[j], acc.at[didx.at[b].at[j]],
                        ssem).wait()

            @pl.when(g < gpt)
            def _():
                base = (gbase + g) * KW
                pltpu.async_copy(src_ref.at[pl.ds(base, KW)],
                                 sidx.at[b], isem[b])
                pltpu.async_copy(dst_ref.at[pl.ds(base, KW)],
                                 didx.at[b], isem[b])

            @pl.when(jnp.logical_and(g >= 1, g <= gpt))
            def _():
                base = (gbase + g - 1) * KW
                pltpu.make_async_copy(src_ref.at[pl.ds(base, KW)],
                                      sidx.at[b1], isem[b1]).wait()
                pltpu.make_async_copy(dst_ref.at[pl.ds(base, KW)],
                                      didx.at[b1], isem[b1]).wait()
                for j in range(KW):
                    pltpu.async_copy(gsrc_ref.at[sidx.at[b1].at[j]],
                                     rows.at[b1].at[j], gsem[b1])

            @pl.when(jnp.logical_and(g >= 2, g <= gpt + 1))
            def _():
                for j in range(KW):
                    pltpu.make_async_copy(
                        gsrc_ref.at[sidx.at[b2].at[j]],
                        rows.at[b2].at[j], gsem[b2]).wait()
                for j in range(KW):
                    pltpu.async_copy(rows.at[b2].at[j],
                                     acc.at[didx.at[b2].at[j]],
                                     ssem, add=True)
        return 0
    lax.fori_loop(0, gpt // 3 + 1, outer, 0)


def _l1_body(src_ref, dst_ref, nout_ref, nin_ref,
             acc, didx, rows, accb, zbuf,
             isem0, isem1, isem2, gsem0, gsem1, gsem2):
    cid = lax.axis_index("c")
    tid = lax.axis_index("s")
    isem = [isem0, isem1, isem2]
    gsem = [gsem0, gsem1, gsem2]
    _fill_zbuf(zbuf, RCH)
    _zero_acc(acc, zbuf, tid)
    plsc.subcore_barrier()

    @pl.when(cid == 0)
    def _():
        _deg_ring(src_ref, acc, didx, rows, isem, gsem, tid, GPT1)

    @pl.when(cid == 1)
    def _():
        _deg_ring(dst_ref, acc, didx, rows, isem, gsem, tid, GPT1)
    plsc.subcore_barrier()

    def norm_phase(out_ref):
        def ch(c, _):
            base = tid * RPT + c * RCH
            pltpu.sync_copy(acc.at[pl.ds(base, RCH)], accb)

            def row(i, _):
                accb[i, :] = _rsqrt16(jnp.maximum(accb[i, :], 1.0))
                return 0
            lax.fori_loop(0, RCH, row, 0)
            pltpu.sync_copy(accb, out_ref.at[pl.ds(base, RCH)])
            return 0
        lax.fori_loop(0, NCH, ch, 0)

    @pl.when(cid == 0)
    def _():
        norm_phase(nout_ref)

    @pl.when(cid == 1)
    def _():
        norm_phase(nin_ref)


_l1 = functools.partial(
    pl.kernel,
    out_type=(jax.ShapeDtypeStruct((N_P, C), _F32),
              jax.ShapeDtypeStruct((N_P, C), _F32)),
    mesh=_MESH, compiler_params=_PARAMS,
    scratch_types=[
        pltpu.VMEM_SHARED((N_P, C), _F32),
        pltpu.VMEM((3, KW, 128), jnp.int32),
        pltpu.VMEM((3, KW, 128, C), _F32),
        pltpu.VMEM((RCH, C), _F32),
        pltpu.VMEM((RCH, C), _F32),
        pltpu.SemaphoreType.DMA, pltpu.SemaphoreType.DMA,
        pltpu.SemaphoreType.DMA, pltpu.SemaphoreType.DMA,
        pltpu.SemaphoreType.DMA, pltpu.SemaphoreType.DMA,
    ],
)(_l1_body)


def _l2_body(nout_ref, nin_ref, h0_ref, nio_ref, g0_ref,
             ab, bb, cb, sem0, sem1, sem2):
    ft = lax.axis_index("c") * NS + lax.axis_index("s")

    def ch(c, _):
        base = ft * RPT32 + c * RCH2
        d0 = pltpu.async_copy(nout_ref.at[pl.ds(base, RCH2)], ab, sem0)
        d1 = pltpu.async_copy(nin_ref.at[pl.ds(base, RCH2)], bb, sem1)
        d2 = pltpu.async_copy(h0_ref.at[pl.ds(base, RCH2)], cb, sem2)
        d0.wait()
        d1.wait()
        d2.wait()

        def row(i, _):
            cb[i, :] = cb[i, :] * ab[i, :]
            bb[i, :] = bb[i, :] * ab[i, :]
            return 0
        lax.fori_loop(0, RCH2, row, 0)
        full = base + RCH2 <= N
        part = jnp.logical_and(base < N, jnp.logical_not(full))

        @pl.when(full)
        def _():
            pltpu.sync_copy(cb, g0_ref.at[pl.ds(base, RCH2)])
            pltpu.sync_copy(bb, nio_ref.at[pl.ds(base, RCH2)])

        @pl.when(part)
        def _():
            pltpu.sync_copy(cb.at[pl.ds(0, TAIL2)],
                            g0_ref.at[pl.ds(base, TAIL2)])
            pltpu.sync_copy(bb.at[pl.ds(0, TAIL2)],
                            nio_ref.at[pl.ds(base, TAIL2)])
        return 0
    lax.fori_loop(0, NCH2, ch, 0)


_l2 = functools.partial(
    pl.kernel,
    out_type=(jax.ShapeDtypeStruct((N_P, C), _F32),
              jax.ShapeDtypeStruct((N_P, C), _F32)),
    mesh=_MESH, compiler_params=_PARAMS,
    scratch_types=[
        pltpu.VMEM((RCH2, C), _F32), pltpu.VMEM((RCH2, C), _F32),
        pltpu.VMEM((RCH2, C), _F32),
        pltpu.SemaphoreType.DMA, pltpu.SemaphoreType.DMA,
        pltpu.SemaphoreType.DMA,
    ],
)(_l2_body)


def _lh_body(src_ref, dst_ref, gsrc_ref, accp_ref,
             acc, sidx, didx, rows, zbuf,
             isem0, isem1, isem2, gsem0, gsem1, gsem2, ssem):
    cid = lax.axis_index("c")
    tid = lax.axis_index("s")
    isem = [isem0, isem1, isem2]
    gsem = [gsem0, gsem1, gsem2]
    gpt = jnp.where(cid == 0, GPT_A, GPT_B)
    gbase = jnp.where(cid == 0, tid * GPT_A, NS * GPT_A + tid * GPT_B)
    _fill_zbuf(zbuf, RCH)
    _zero_acc(acc, zbuf, tid)
    plsc.subcore_barrier()
    _edge_ring(src_ref, dst_ref, gsrc_ref, acc, sidx, didx, rows,
               isem, gsem, ssem, gbase, gpt)
    plsc.subcore_barrier()
    pltpu.sync_copy(acc.at[pl.ds(tid * RPT, RPT)],
                    accp_ref.at[cid].at[pl.ds(tid * RPT, RPT)])


_lh = functools.partial(
    pl.kernel,
    out_type=jax.ShapeDtypeStruct((2, N_P, C), _F32),
    mesh=_MESH, compiler_params=_PARAMS,
    scratch_types=[
        pltpu.VMEM_SHARED((N_P, C), _F32),
        pltpu.VMEM((3, KW, 128), jnp.int32),
        pltpu.VMEM((3, KW, 128), jnp.int32),
        pltpu.VMEM((3, KW, 128, C), _F32),
        pltpu.VMEM((RCH, C), _F32),
        pltpu.SemaphoreType.DMA, pltpu.SemaphoreType.DMA,
        pltpu.SemaphoreType.DMA, pltpu.SemaphoreType.DMA,
        pltpu.SemaphoreType.DMA, pltpu.SemaphoreType.DMA,
        pltpu.SemaphoreType.DMA,
    ],
)(_lh_body)


def _combine_body(final, accp_ref, norm_ref, aux_ref, out_ref,
                  ab, bb, cb, db, sem0, sem1, sem2, sem3):
    ft = lax.axis_index("c") * NS + lax.axis_index("s")

    def ch(c, _):
        base = ft * RPT32 + c * RCH2
        d0 = pltpu.async_copy(accp_ref.at[0].at[pl.ds(base, RCH2)], ab, sem0)
        d1 = pltpu.async_copy(accp_ref.at[1].at[pl.ds(base, RCH2)], bb, sem1)
        d2 = pltpu.async_copy(norm_ref.at[pl.ds(base, RCH2)], cb, sem2)
        d3 = pltpu.async_copy(aux_ref.at[pl.ds(base, RCH2)], db, sem3)
        d0.wait()
        d1.wait()
        d2.wait()
        d3.wait()

        def row(i, _):
            ab[i, :] = ((1.0 - ALPHA) * cb[i, :]) * (ab[i, :] + bb[i, :]) \
                + ALPHA * db[i, :]
            return 0
        lax.fori_loop(0, RCH2, row, 0)
        full = base + RCH2 <= N
        part = jnp.logical_and(base < N, jnp.logical_not(full))

        @pl.when(full)
        def _():
            pltpu.sync_copy(ab, out_ref.at[pl.ds(base, RCH2)])

        @pl.when(part)
        def _():
            pltpu.sync_copy(ab.at[pl.ds(0, TAIL2)],
                            out_ref.at[pl.ds(base, TAIL2)])
        return 0
    lax.fori_loop(0, NCH2, ch, 0)


def _mk_combine(final):
    shape = (N, C) if final else (N_P, C)
    return functools.partial(
        pl.kernel,
        out_type=jax.ShapeDtypeStruct(shape, _F32),
        mesh=_MESH, compiler_params=_PARAMS,
        scratch_types=[
            pltpu.VMEM((RCH2, C), _F32), pltpu.VMEM((RCH2, C), _F32),
            pltpu.VMEM((RCH2, C), _F32), pltpu.VMEM((RCH2, C), _F32),
            pltpu.SemaphoreType.DMA, pltpu.SemaphoreType.DMA,
            pltpu.SemaphoreType.DMA, pltpu.SemaphoreType.DMA,
        ],
    )(functools.partial(_combine_body, final))


_lc = _mk_combine(False)
_lf = _mk_combine(True)


def _mm_body(x_ref, wt_ref, b_ref, o_ref):
    o_ref[...] = jnp.dot(x_ref[...], wt_ref[...],
                         preferred_element_type=_F32) + b_ref[...]


def _mm(x, wt, b2):
    bm = 512
    return pl.pallas_call(
        _mm_body,
        grid=(N_P // bm,),
        in_specs=[pl.BlockSpec((bm, F), lambda m: (m, 0)),
                  pl.BlockSpec((F, C), lambda m: (0, 0)),
                  pl.BlockSpec((1, C), lambda m: (0, 0))],
        out_specs=pl.BlockSpec((bm, C), lambda m: (m, 0)),
        out_shape=jax.ShapeDtypeStruct((N_P, C), _F32),
    )(x, wt, b2)


def kernel(in_feat, edge_index, W, b):
    src = edge_index[0].astype(jnp.int32)
    dst = edge_index[1].astype(jnp.int32)
    pad = jnp.full((E_P - E,), PAD_IDX, jnp.int32)
    src2 = jnp.concatenate([src, pad]).reshape(IDX_ROWS, 128)
    dst2 = jnp.concatenate([dst, pad]).reshape(IDX_ROWS, 128)
    xp = jnp.pad(in_feat, ((0, N_P - N), (0, 0)))
    h0 = _mm(xp, W.T, b[None, :])
    nout, nin = _l1(src2, dst2)
    nio, g0 = _l2(nout, nin, h0)
    g = g0
    for hop in range(3):
        accp = _lh(src2, dst2, g)
        if hop < 2:
            g = _lc(accp, nio, g0)
        else:
            return _lf(accp, nin, h0)
